# baseline (speedup 1.0000x reference)
import functools

import jax
import jax.numpy as jnp
from jax import lax
from jax.experimental import pallas as pl
from jax.experimental.pallas import tpu as pltpu
from jax.experimental.pallas import tpu_sc as plsc

B, LEN, CH = 4, 4096, 1024
G = 4
LO = LEN // G
NC, NS = 2, 16
NW = NC * NS
PAIRS = B * G
HALVES = NW // PAIRS
W = LO // HALVES


def _body(in_hbm, out_hbm):
    wid = lax.axis_index("s") * NC + lax.axis_index("c")
    pair = wid // HALVES
    half = wid % HALVES
    b = pair // G
    hi = pair % G
    lo0 = half * W
    pltpu.sync_copy(
        in_hbm.at[b, pl.ds(lo0, W), hi],
        out_hbm.at[b, hi, pl.ds(lo0, W)],
    )


_shuffle = pl.kernel(
    _body,
    out_type=jax.ShapeDtypeStruct((B, G, LO, CH), jnp.float32),
    mesh=plsc.VectorSubcoreMesh(core_axis_name="c", subcore_axis_name="s"),
)


def kernel(inputs):
    in4 = inputs.reshape(B, LO, G, CH)
    out4 = _shuffle(in4)
    return out4.reshape(B, LEN, CH)


# speedup vs baseline: 29.8023x; 29.8023x over previous
import functools

import jax
import jax.numpy as jnp
import numpy as np
from jax import lax
from jax.experimental import pallas as pl
from jax.experimental.pallas import tpu as pltpu
from jax.experimental.pallas import tpu_sc as plsc

B, LEN, CH = 4, 4096, 1024
ROWS = B * LEN
NC, NS = 2, 16
NW = NC * NS
WPW = ROWS // NW
CHUNK = 64
NCHUNK = WPW // CHUNK

_i = np.arange(LEN)
_perm = (_i % (LEN // 4)) * 4 + _i // (LEN // 4)
_SRC = (np.arange(ROWS) // LEN * LEN + np.tile(_perm, B)).astype(np.int32)


def _body(in_hbm, idx_hbm, out_hbm, idx_v, rows_v, sem):
    wid = lax.axis_index("s") * NC + lax.axis_index("c")
    base = wid * WPW
    pltpu.sync_copy(idx_hbm.at[pl.ds(base, WPW)], idx_v)
    for k in range(NCHUNK):
        pltpu.async_copy(
            in_hbm.at[idx_v.at[pl.ds(k * CHUNK, CHUNK)]], rows_v, sem
        ).wait()
        pltpu.sync_copy(rows_v, out_hbm.at[pl.ds(base + k * CHUNK, CHUNK)])


_shuffle = pl.kernel(
    _body,
    out_type=jax.ShapeDtypeStruct((ROWS, CH), jnp.float32),
    mesh=plsc.VectorSubcoreMesh(core_axis_name="c", subcore_axis_name="s"),
    scratch_types=[
        pltpu.VMEM((WPW,), jnp.int32),
        pltpu.VMEM((CHUNK, CH), jnp.float32),
        pltpu.SemaphoreType.DMA,
    ],
)


def kernel(inputs):
    in_flat = inputs.reshape(ROWS, CH)
    out_flat = _shuffle(in_flat, jnp.asarray(_SRC))
    return out_flat.reshape(B, LEN, CH)


# speedup vs baseline: 30.6622x; 1.0289x over previous
import functools

import jax
import jax.numpy as jnp
import numpy as np
from jax import lax
from jax.experimental import pallas as pl
from jax.experimental.pallas import tpu as pltpu
from jax.experimental.pallas import tpu_sc as plsc

B, LEN, CH = 4, 4096, 1024
ROWS = B * LEN
NC, NS = 2, 16
NW = NC * NS
WPW = ROWS // NW
CHUNK = 32
NCHUNK = WPW // CHUNK

_i = np.arange(LEN)
_perm = (_i % (LEN // 4)) * 4 + _i // (LEN // 4)
_SRC = (np.arange(ROWS) // LEN * LEN + np.tile(_perm, B)).astype(np.int32)


def _body(in_hbm, idx_hbm, out_hbm, idx_v, rows_v, gsem, ssem):
    wid = lax.axis_index("s") * NC + lax.axis_index("c")
    base = wid * WPW
    pltpu.sync_copy(idx_hbm.at[pl.ds(base, WPW)], idx_v)

    def gather(k, buf):
        return pltpu.async_copy(
            in_hbm.at[idx_v.at[pl.ds(k * CHUNK, CHUNK)]],
            rows_v.at[buf],
            gsem.at[buf],
        )

    def store(k, buf):
        return pltpu.async_copy(
            rows_v.at[buf],
            out_hbm.at[pl.ds(base + k * CHUNK, CHUNK)],
            ssem.at[buf],
        )

    g = [gather(0, 0), None]
    s = [None, None]
    for k in range(NCHUNK):
        cur, nxt = k & 1, (k + 1) & 1
        g[cur].wait()
        if k + 1 < NCHUNK:
            if s[nxt] is not None:
                s[nxt].wait()
            g[nxt] = gather(k + 1, nxt)
        s[cur] = store(k, cur)
    s[(NCHUNK - 1) & 1].wait()


_shuffle = pl.kernel(
    _body,
    out_type=jax.ShapeDtypeStruct((ROWS, CH), jnp.float32),
    mesh=plsc.VectorSubcoreMesh(core_axis_name="c", subcore_axis_name="s"),
    scratch_types=[
        pltpu.VMEM((WPW,), jnp.int32),
        pltpu.VMEM((2, CHUNK, CH), jnp.float32),
        pltpu.SemaphoreType.DMA((2,)),
        pltpu.SemaphoreType.DMA((2,)),
    ],
)


def kernel(inputs):
    in_flat = inputs.reshape(ROWS, CH)
    out_flat = _shuffle(in_flat, jnp.asarray(_SRC))
    return out_flat.reshape(B, LEN, CH)
